# baseline (speedup 1.0000x reference)
import jax
import jax.numpy as jnp
from jax.experimental import pallas as pl
from jax.experimental.pallas import tpu as pltpu

_MAX_TIME_MS = 600000.0
_ROW_BLK = 1024


def _fma_kernel(t_ref, w_ref, b_ref, o_ref):
    w_scaled = w_ref[...] * (1.0 / _MAX_TIME_MS)
    o_ref[...] = t_ref[...] * w_scaled + b_ref[...]


def kernel(timestamps, W, b):
    S, B, _ = timestamps.shape
    D = b.shape[0]
    n = S * B
    t2 = timestamps.reshape(n, 1)
    w_row = W.reshape(1, D)
    b_row = b.reshape(1, D)

    grid = (n // _ROW_BLK,)
    out = pl.pallas_call(
        _fma_kernel,
        grid=grid,
        in_specs=[
            pl.BlockSpec((_ROW_BLK, 1), lambda i: (i, 0)),
            pl.BlockSpec((1, D), lambda i: (0, 0)),
            pl.BlockSpec((1, D), lambda i: (0, 0)),
        ],
        out_specs=pl.BlockSpec((_ROW_BLK, D), lambda i: (i, 0)),
        out_shape=jax.ShapeDtypeStruct((n, D), jnp.float32),
        compiler_params=pltpu.CompilerParams(
            dimension_semantics=("arbitrary",),
        ),
    )(t2, w_row, b_row)
    return out.reshape(S, B, D)
